# baseline (speedup 1.0000x reference)
import jax
import jax.numpy as jnp
from jax import lax
from jax.experimental import pallas as pl
from jax.experimental.pallas import tpu as pltpu

N = 4096
D = 512
C = 128
BLK = 512
GBLK = 1024
TBLK = 1024
ALPHA = 0.2
_NT = (((1,), (1,)), ((), ()))
_PAR = pltpu.CompilerParams(dimension_semantics=("parallel",))


def _bf16(x):
    return x.astype(jnp.bfloat16)


def _proj_body(x_ref, enc_ref, watt_ref, xproj_ref, wh_ref):
    xp = jnp.dot(_bf16(x_ref[...]), enc_ref[...],
                 preferred_element_type=jnp.float32)
    wh = jnp.dot(_bf16(xp), watt_ref[...], preferred_element_type=jnp.float32)
    xproj_ref[...] = _bf16(xp)
    wh_ref[...] = _bf16(wh)


def _attn_body(a_ref, wh_ref, xproj_ref, w1_ref, adj_ref, z1_ref, adjb_ref):
    adjb_ref[...] = _bf16(adj_ref[...])
    i = pl.program_id(0)
    wh = wh_ref[...]
    wh_blk = wh_ref[pl.ds(i * BLK, BLK), :]
    a8 = a_ref[...]
    eblk = lax.dot_general(wh_blk, a8, _NT, preferred_element_type=jnp.float32)
    erow = lax.dot_general(a8, wh, _NT, preferred_element_type=jnp.float32)
    e1 = eblk[:, 0:1]
    e2 = erow[1:2, :]
    m2 = jnp.max(e2)
    t = e1 + m2
    m = jnp.where(t >= 0, t, ALPHA * t)
    p = jnp.exp(t - m)
    q = jnp.exp(ALPHA * t - m)
    u = jnp.exp(e2 - m2)
    v = jnp.exp(ALPHA * (e2 - m2))
    s = _bf16(jnp.where(e1 + e2 > 0, p * u, q * v))
    ones8 = jnp.ones((8, N), jnp.bfloat16)
    z = lax.dot_general(s, ones8, _NT,
                        preferred_element_type=jnp.float32)[:, 0:1]
    acc = jnp.dot(s, wh, preferred_element_type=jnp.float32)
    x_ent = xproj_ref[...].astype(jnp.float32) * (acc / z)
    z1_ref[...] = _bf16(
        jnp.dot(_bf16(x_ent), w1_ref[...], preferred_element_type=jnp.float32))


def _gcn1_body(adj_ref, z1_ref, b1_ref, w2_ref, w3_ref, t1_ref):
    w23 = _bf16(jnp.dot(w2_ref[...], w3_ref[...],
                        preferred_element_type=jnp.float32))
    y1 = jnp.dot(adj_ref[...], z1_ref[...], preferred_element_type=jnp.float32)
    y1 = jnp.maximum(y1 + b1_ref[...], 0.0)
    t1_ref[...] = _bf16(
        jnp.dot(_bf16(y1), w23, preferred_element_type=jnp.float32))


def _tail2_body(adj_ref, t1_ref, b2_ref, w3_ref, z3_ref):
    b2w3 = jnp.dot(b2_ref[...], w3_ref[...],
                   preferred_element_type=jnp.float32)
    y = jnp.dot(adj_ref[...], t1_ref[...], preferred_element_type=jnp.float32)
    z3_ref[...] = _bf16(y + b2w3)


def _gcn3_body(adj_ref, z3_ref, b3_ref, out_ref):
    y3 = jnp.dot(adj_ref[...], z3_ref[...], preferred_element_type=jnp.float32)
    y3 = y3 + b3_ref[...]
    m = jnp.max(y3, axis=1, keepdims=True)
    s = y3 - m
    lse = jnp.log(jnp.sum(jnp.exp(s), axis=1, keepdims=True))
    out_ref[...] = s - lse


def _row_blocked(d):
    return pl.BlockSpec((BLK, d), lambda i: (i, 0))


def _grow_blocked(d):
    return pl.BlockSpec((GBLK, d), lambda i: (i, 0))


def _tail_blocked(d):
    return pl.BlockSpec((TBLK, d), lambda i: (i, 0))


def _whole(r, c):
    return pl.BlockSpec((r, c), lambda i: (0, 0))


def kernel(x_org, adj, encoder1, W_att, a_att, gc1_W, gc1_b, gc2_W, gc2_b,
           gc3_W, gc3_b):
    grid = (N // BLK,)
    ggrid = (N // GBLK,)
    tgrid = (N // TBLK,)
    a_pair = jnp.zeros((8, D), jnp.bfloat16).at[0:2].set(
        _bf16(a_att.reshape(2, D)))

    xproj, wh = pl.pallas_call(
        _proj_body,
        grid=grid,
        compiler_params=_PAR,
        in_specs=[_row_blocked(D), _whole(D, D), _whole(D, D)],
        out_specs=[_row_blocked(D), _row_blocked(D)],
        out_shape=[jax.ShapeDtypeStruct((N, D), jnp.bfloat16)] * 2,
    )(x_org, _bf16(encoder1), _bf16(W_att))

    z1, adjb = pl.pallas_call(
        _attn_body,
        grid=grid,
        compiler_params=_PAR,
        in_specs=[_whole(8, D), _whole(N, D), _row_blocked(D), _whole(D, D),
                  _row_blocked(N)],
        out_specs=[_row_blocked(D), _row_blocked(N)],
        out_shape=[jax.ShapeDtypeStruct((N, D), jnp.bfloat16),
                   jax.ShapeDtypeStruct((N, N), jnp.bfloat16)],
    )(a_pair, wh, xproj, _bf16(gc1_W), adj)

    t1 = pl.pallas_call(
        _gcn1_body,
        grid=ggrid,
        compiler_params=_PAR,
        in_specs=[_grow_blocked(N), _whole(N, D), _whole(1, D), _whole(D, D),
                  _whole(D, C)],
        out_specs=_grow_blocked(C),
        out_shape=jax.ShapeDtypeStruct((N, C), jnp.bfloat16),
    )(adjb, z1, gc1_b.reshape(1, D), _bf16(gc2_W), _bf16(gc3_W))

    z3 = pl.pallas_call(
        _tail2_body,
        grid=tgrid,
        compiler_params=_PAR,
        in_specs=[_tail_blocked(N), _whole(N, C), _whole(1, D), _whole(D, C)],
        out_specs=_tail_blocked(C),
        out_shape=jax.ShapeDtypeStruct((N, C), jnp.bfloat16),
    )(adjb, t1, gc2_b.reshape(1, D), _bf16(gc3_W))

    out = pl.pallas_call(
        _gcn3_body,
        grid=tgrid,
        compiler_params=_PAR,
        in_specs=[_tail_blocked(N), _whole(N, C), _whole(1, C)],
        out_specs=_tail_blocked(C),
        out_shape=jax.ShapeDtypeStruct((N, C), jnp.float32),
    )(adjb, z3, gc3_b.reshape(1, C))

    return out


# speedup vs baseline: 1.0343x; 1.0343x over previous
import jax
import jax.numpy as jnp
from jax import lax
from jax.experimental import pallas as pl
from jax.experimental.pallas import tpu as pltpu

N = 4096
D = 512
C = 128
BLK = 512
GBLK = 1024
TBLK = 1024
ALPHA = 0.2
_NT = (((1,), (1,)), ((), ()))
_PAR = pltpu.CompilerParams(dimension_semantics=("parallel",))


def _bf16(x):
    return x.astype(jnp.bfloat16)


def _proj_body(x_ref, enc_ref, watt_ref, xproj_ref, wh_ref):
    xp = jnp.dot(_bf16(x_ref[...]), enc_ref[...],
                 preferred_element_type=jnp.float32)
    wh = jnp.dot(_bf16(xp), watt_ref[...], preferred_element_type=jnp.float32)
    xproj_ref[...] = _bf16(xp)
    wh_ref[...] = _bf16(wh)


def _attn_body(a_ref, wh_ref, xproj_ref, w1_ref, adj_ref, z1_ref, adjb_ref):
    adjb_ref[...] = _bf16(adj_ref[...])
    i = pl.program_id(0)
    wh = wh_ref[...]
    wh_blk = wh_ref[pl.ds(i * BLK, BLK), :]
    a8 = a_ref[...]
    eblk = lax.dot_general(wh_blk, a8, _NT, preferred_element_type=jnp.float32)
    erow = lax.dot_general(a8, wh, _NT, preferred_element_type=jnp.float32)
    e1 = eblk[:, 0:1]
    e2 = erow[1:2, :]
    m2 = jnp.max(e2)
    t = e1 + m2
    m = jnp.where(t >= 0, t, ALPHA * t)
    p = jnp.exp(t - m)
    q = jnp.exp(ALPHA * t - m)
    u = jnp.exp(e2 - m2)
    v = jnp.exp(ALPHA * (e2 - m2))
    s = jnp.where(e1 + e2 > 0, p * u, q * v)
    z = jnp.sum(s, axis=1, keepdims=True)
    acc = jnp.dot(_bf16(s), wh, preferred_element_type=jnp.float32)
    x_ent = xproj_ref[...].astype(jnp.float32) * (acc / z)
    z1_ref[...] = _bf16(
        jnp.dot(_bf16(x_ent), w1_ref[...], preferred_element_type=jnp.float32))


def _gcn1_body(adj_ref, z1_ref, b1_ref, w2_ref, w3_ref, t1_ref):
    w23 = _bf16(jnp.dot(w2_ref[...], w3_ref[...],
                        preferred_element_type=jnp.float32))
    y1 = jnp.dot(adj_ref[...], z1_ref[...], preferred_element_type=jnp.float32)
    y1 = jnp.maximum(y1 + b1_ref[...], 0.0)
    t1_ref[...] = _bf16(
        jnp.dot(_bf16(y1), w23, preferred_element_type=jnp.float32))


def _tail2_body(adj_ref, t1_ref, b2_ref, w3_ref, z3_ref):
    b2w3 = jnp.dot(b2_ref[...], w3_ref[...],
                   preferred_element_type=jnp.float32)
    y = jnp.dot(adj_ref[...], t1_ref[...], preferred_element_type=jnp.float32)
    z3_ref[...] = _bf16(y + b2w3)


def _gcn3_body(adj_ref, z3_ref, b3_ref, out_ref):
    y3 = jnp.dot(adj_ref[...], z3_ref[...], preferred_element_type=jnp.float32)
    y3 = y3 + b3_ref[...]
    m = jnp.max(y3, axis=1, keepdims=True)
    s = y3 - m
    lse = jnp.log(jnp.sum(jnp.exp(s), axis=1, keepdims=True))
    out_ref[...] = s - lse


def _row_blocked(d):
    return pl.BlockSpec((BLK, d), lambda i: (i, 0))


def _grow_blocked(d):
    return pl.BlockSpec((GBLK, d), lambda i: (i, 0))


def _tail_blocked(d):
    return pl.BlockSpec((TBLK, d), lambda i: (i, 0))


def _whole(r, c):
    return pl.BlockSpec((r, c), lambda i: (0, 0))


def kernel(x_org, adj, encoder1, W_att, a_att, gc1_W, gc1_b, gc2_W, gc2_b,
           gc3_W, gc3_b):
    grid = (N // BLK,)
    ggrid = (N // GBLK,)
    tgrid = (N // TBLK,)
    a_pair = jnp.zeros((8, D), jnp.bfloat16).at[0:2].set(
        _bf16(a_att.reshape(2, D)))

    xproj, wh = pl.pallas_call(
        _proj_body,
        grid=grid,
        compiler_params=_PAR,
        in_specs=[_row_blocked(D), _whole(D, D), _whole(D, D)],
        out_specs=[_row_blocked(D), _row_blocked(D)],
        out_shape=[jax.ShapeDtypeStruct((N, D), jnp.bfloat16)] * 2,
    )(x_org, _bf16(encoder1), _bf16(W_att))

    z1, adjb = pl.pallas_call(
        _attn_body,
        grid=grid,
        compiler_params=_PAR,
        in_specs=[_whole(8, D), _whole(N, D), _row_blocked(D), _whole(D, D),
                  _row_blocked(N)],
        out_specs=[_row_blocked(D), _row_blocked(N)],
        out_shape=[jax.ShapeDtypeStruct((N, D), jnp.bfloat16),
                   jax.ShapeDtypeStruct((N, N), jnp.bfloat16)],
    )(a_pair, wh, xproj, _bf16(gc1_W), adj)

    t1 = pl.pallas_call(
        _gcn1_body,
        grid=ggrid,
        compiler_params=_PAR,
        in_specs=[_grow_blocked(N), _whole(N, D), _whole(1, D), _whole(D, D),
                  _whole(D, C)],
        out_specs=_grow_blocked(C),
        out_shape=jax.ShapeDtypeStruct((N, C), jnp.bfloat16),
    )(adjb, z1, gc1_b.reshape(1, D), _bf16(gc2_W), _bf16(gc3_W))

    z3 = pl.pallas_call(
        _tail2_body,
        grid=tgrid,
        compiler_params=_PAR,
        in_specs=[_tail_blocked(N), _whole(N, C), _whole(1, D), _whole(D, C)],
        out_specs=_tail_blocked(C),
        out_shape=jax.ShapeDtypeStruct((N, C), jnp.bfloat16),
    )(adjb, t1, gc2_b.reshape(1, D), _bf16(gc3_W))

    out = pl.pallas_call(
        _gcn3_body,
        grid=tgrid,
        compiler_params=_PAR,
        in_specs=[_tail_blocked(N), _whole(N, C), _whole(1, C)],
        out_specs=_tail_blocked(C),
        out_shape=jax.ShapeDtypeStruct((N, C), jnp.float32),
    )(adjb, z3, gc3_b.reshape(1, C))

    return out
